# baseline (speedup 1.0000x reference)
import jax
import jax.numpy as jnp
from jax.experimental import pallas as pl
from jax.experimental.pallas import tpu as pltpu
from jax.experimental.pallas import tpu_sc as plsc

BB = 8
K = 4


def _sc_gather(idx, table):
    Bc, T = idx.shape
    V, D = table.shape
    mesh = plsc.VectorSubcoreMesh(core_axis_name="c", subcore_axis_name="s")

    @pl.kernel(
        out_type=jax.ShapeDtypeStruct((Bc, T, D), table.dtype),
        mesh=mesh,
        scratch_types=[pltpu.SemaphoreType.DMA],
    )
    def gather_kernel(table_hbm, idx_hbm, out_hbm, sem):
        def body(i_vmem, o_vmem):
            copies = [
                pltpu.async_copy(table_hbm.at[i_vmem.at[j]], o_vmem.at[j], sem)
                for j in range(BB)
            ]
            for cp in copies:
                cp.wait()

        pltpu.emit_pipeline(
            body,
            grid=(Bc // BB,),
            in_specs=[pl.BlockSpec((BB, T), lambda i: (i, 0))],
            out_specs=[pl.BlockSpec((BB, T, D), lambda i: (i, 0, 0))],
            core_axis_name=("c", "s"),
            dimension_semantics=(pltpu.PARALLEL,),
        )(idx_hbm, out_hbm)

    return gather_kernel(table, idx)


def kernel(value, table):
    B, T = value.shape
    idx = value.astype(jnp.int32)
    chunk = B // K
    parts = [_sc_gather(idx[k * chunk:(k + 1) * chunk], table) for k in range(K)]
    return jnp.concatenate(parts, axis=0)


# speedup vs baseline: 1.1276x; 1.1276x over previous
import jax
import jax.numpy as jnp
from jax.experimental import pallas as pl
from jax.experimental.pallas import tpu as pltpu
from jax.experimental.pallas import tpu_sc as plsc

BB = 8
K = 4
TCB = 32


def _sc_gather_chunk(idx, table, k, chunk_b):
    B, T = idx.shape
    V, D = table.shape
    mesh = plsc.VectorSubcoreMesh(core_axis_name="c", subcore_axis_name="s")
    base = k * (chunk_b // BB)

    @pl.kernel(
        out_type=jax.ShapeDtypeStruct((chunk_b * T, D), table.dtype),
        mesh=mesh,
        scratch_types=[pltpu.SemaphoreType.DMA],
    )
    def gather_kernel(table_hbm, idx_hbm, out_hbm, sem):
        def body(i_vmem, o_vmem):
            copies = [
                pltpu.async_copy(
                    table_hbm.at[i_vmem.at[j]],
                    o_vmem.at[pl.ds(j * T, T)],
                    sem,
                )
                for j in range(BB)
            ]
            for cp in copies:
                cp.wait()

        pltpu.emit_pipeline(
            body,
            grid=(chunk_b // BB,),
            in_specs=[pl.BlockSpec((BB, T), lambda i: (base + i, 0))],
            out_specs=[pl.BlockSpec((BB * T, D), lambda i: (i, 0))],
            core_axis_name=("c", "s"),
            dimension_semantics=(pltpu.PARALLEL,),
        )(idx_hbm, out_hbm)

    return gather_kernel(table, idx)


def _tc_assemble(acc, rows, k, chunk_b, B, T, D):
    grid = chunk_b // TCB
    base = k * grid

    def body(*refs):
        in_ref, o_ref = refs[-2], refs[-1]
        for j in range(TCB):
            o_ref[j] = in_ref[pl.ds(j * T, T), :]

    if acc is None:
        in_specs = [pl.BlockSpec((TCB * T, D), lambda i: (i, 0))]
        aliases = {}
        args = (rows,)
    else:
        in_specs = [
            pl.BlockSpec(memory_space=pl.ANY),
            pl.BlockSpec((TCB * T, D), lambda i: (i, 0)),
        ]
        aliases = {0: 0}
        args = (acc, rows)

    return pl.pallas_call(
        body,
        grid=(grid,),
        in_specs=in_specs,
        out_specs=pl.BlockSpec((TCB, T, D), lambda i: (base + i, 0, 0)),
        out_shape=jax.ShapeDtypeStruct((B, T, D), jnp.float32),
        input_output_aliases=aliases,
    )(*args)


def kernel(value, table):
    B, T = value.shape
    V, D = table.shape
    idx = value.astype(jnp.int32)
    chunk_b = B // K

    parts = [_sc_gather_chunk(idx, table, k, chunk_b) for k in range(K)]
    acc = None
    for k in range(K):
        acc = _tc_assemble(acc, parts[k], k, chunk_b, B, T, D)
    return acc
